# baseline (speedup 1.0000x reference)
import functools

import jax
import jax.numpy as jnp
from jax import lax
from jax.experimental import pallas as pl
from jax.experimental.pallas import tpu as pltpu

_NB = 16


def _pair_mlp_body(raw_ref, w0_ref, b0_ref, w1_ref, b1_ref, w2_ref, b2_ref,
                   out_ref):
    h = jnp.maximum(
        jnp.dot(raw_ref[...], w0_ref[...],
                preferred_element_type=jnp.float32) + b0_ref[...], 0.0)
    h = jnp.maximum(
        jnp.dot(h, w1_ref[...],
                preferred_element_type=jnp.float32) + b1_ref[...], 0.0)
    h = jnp.maximum(
        jnp.dot(h, w2_ref[...],
                preferred_element_type=jnp.float32) + b2_ref[...], 0.0)
    out_ref[...] = h


def _pair_mlp(raw, w0, b0, w1, b1, w2, b2, tile=6400):
    E = raw.shape[0]
    grid = E // tile
    b0 = b0.reshape(1, -1)
    b1 = b1.reshape(1, -1)
    b2 = b2.reshape(1, -1)
    return pl.pallas_call(
        _pair_mlp_body,
        grid=(grid,),
        in_specs=[
            pl.BlockSpec((tile, raw.shape[1]), lambda i: (i, 0)),
            pl.BlockSpec(w0.shape, lambda i: (0, 0)),
            pl.BlockSpec(b0.shape, lambda i: (0, 0)),
            pl.BlockSpec(w1.shape, lambda i: (0, 0)),
            pl.BlockSpec(b1.shape, lambda i: (0, 0)),
            pl.BlockSpec(w2.shape, lambda i: (0, 0)),
            pl.BlockSpec(b2.shape, lambda i: (0, 0)),
        ],
        out_specs=pl.BlockSpec((tile, w2.shape[1]), lambda i: (i, 0)),
        out_shape=jax.ShapeDtypeStruct((E, w2.shape[1]), jnp.float32),
    )(raw, w0, b0, w1, b1, w2, b2)


def _edge_mlp_body(p_ref, cg_ref, ng_ref, w1p_ref, w1c_ref, w1n_ref, b1_ref,
                   w2_ref, b2_ref, out_ref):
    acc = jnp.dot(p_ref[...], w1p_ref[...], preferred_element_type=jnp.float32)
    acc += jnp.dot(cg_ref[...], w1c_ref[...], preferred_element_type=jnp.float32)
    acc += jnp.dot(ng_ref[...], w1n_ref[...], preferred_element_type=jnp.float32)
    h = jnp.maximum(acc + b1_ref[...], 0.0)
    h = jnp.maximum(
        jnp.dot(h, w2_ref[...], preferred_element_type=jnp.float32)
        + b2_ref[...], 0.0)
    out_ref[...] = h


def _edge_mlp(p, cg, ng, w1, b1, w2, b2, out_rows, tile=6400):
    E = p.shape[0]
    grid = E // tile
    w1p, w1c, w1n = w1[0:32], w1[32:64], w1[64:96]
    b1 = b1.reshape(1, -1)
    b2 = b2.reshape(1, -1)
    return pl.pallas_call(
        _edge_mlp_body,
        grid=(grid,),
        in_specs=[
            pl.BlockSpec((tile, 32), lambda i: (i, 0)),
            pl.BlockSpec((tile, 32), lambda i: (i, 0)),
            pl.BlockSpec((tile, 32), lambda i: (i, 0)),
            pl.BlockSpec(w1p.shape, lambda i: (0, 0)),
            pl.BlockSpec(w1c.shape, lambda i: (0, 0)),
            pl.BlockSpec(w1n.shape, lambda i: (0, 0)),
            pl.BlockSpec(b1.shape, lambda i: (0, 0)),
            pl.BlockSpec(w2.shape, lambda i: (0, 0)),
            pl.BlockSpec(b2.shape, lambda i: (0, 0)),
        ],
        out_specs=pl.BlockSpec((tile, 64), lambda i: (i, 0)),
        out_shape=jax.ShapeDtypeStruct((out_rows, 64), jnp.float32),
    )(p, cg, ng, w1p, w1c, w1n, b1, w2, b2)


def _node_fc1_body(x_ref, w_ref, b_ref, out_ref):
    out_ref[...] = jnp.maximum(
        jnp.dot(x_ref[...], w_ref[...], preferred_element_type=jnp.float32)
        + b_ref[...], 0.0)


def _node_fc1(x, w, b):
    N = x.shape[0]
    b = b.reshape(1, -1)
    return pl.pallas_call(
        _node_fc1_body,
        out_shape=jax.ShapeDtypeStruct((N, w.shape[1]), jnp.float32),
    )(x, w, b)


def _node_post_body(x_ref, pooled_ref, w1_ref, b1_ref, w2_ref, b2_ref,
                    wo_ref, bo_ref, out_ref):
    h = jnp.maximum(
        jnp.dot(pooled_ref[...], w1_ref[...],
                preferred_element_type=jnp.float32) + b1_ref[...], 0.0)
    h = jnp.maximum(
        jnp.dot(h, w2_ref[...], preferred_element_type=jnp.float32)
        + b2_ref[...], 0.0)
    refined = jnp.dot(h, wo_ref[...],
                      preferred_element_type=jnp.float32) + bo_ref[...]
    out_ref[...] = jnp.maximum(x_ref[...] + refined, 0.0)


def _node_post(x, pooled, w1, b1, w2, b2, wo, bo):
    N = x.shape[0]
    b1 = b1.reshape(1, -1)
    b2 = b2.reshape(1, -1)
    bo = bo.reshape(1, -1)
    return pl.pallas_call(
        _node_post_body,
        out_shape=jax.ShapeDtypeStruct((N, 128), jnp.float32),
    )(x, pooled, w1, b1, w2, b2, wo, bo)


def kernel(detFeatures, cIdxs, nIdxs, pairFeatRaw,
           pw_W0, pw_b0, pw_W1, pw_b1, pw_W2, pw_b2,
           blk_fc1_W, blk_fc1_b, blk_pw1_W, blk_pw1_b,
           blk_pw2_W, blk_pw2_b, blk_po1_W, blk_po1_b,
           blk_po2_W, blk_po2_b, blk_out_W, blk_out_b):
    N = detFeatures.shape[0]
    E = cIdxs.shape[0]

    p = _pair_mlp(pairFeatRaw, pw_W0, pw_b0, pw_W1, pw_b1, pw_W2, pw_b2)

    x = detFeatures
    for i in range(_NB):
        f1 = _node_fc1(x, blk_fc1_W[i], blk_fc1_b[i])
        cg = jnp.take(f1, cIdxs, axis=0)
        ng = jnp.take(f1, nIdxs, axis=0)
        comb2 = _edge_mlp(p, cg, ng, blk_pw1_W[i], blk_pw1_b[i],
                          blk_pw2_W[i], blk_pw2_b[i], out_rows=E + 512)
        pooled = jax.ops.segment_max(comb2[:E], cIdxs, num_segments=N)
        pooled = jnp.where(jnp.isfinite(pooled), pooled, 0.0)
        x = _node_post(x, pooled, blk_po1_W[i], blk_po1_b[i],
                       blk_po2_W[i], blk_po2_b[i], blk_out_W[i], blk_out_b[i])
    return x
